# baseline (speedup 1.0000x reference)
import functools

import jax
import jax.numpy as jnp
from jax import lax
from jax.experimental import pallas as pl
from jax.experimental.pallas import tpu as pltpu

K = 4


def _mcopy(x_any, o_any, slab, insem, outsem):
    b = pl.program_id(0)
    nb = pl.num_programs(0)
    slot = lax.rem(b, 2)
    C = slab.shape[1]
    CK = C // K

    def in_cp(step, sl, j):
        return pltpu.make_async_copy(
            x_any.at[step, pl.ds(j * CK, CK), :],
            slab.at[sl, pl.ds(j * CK, CK), :],
            insem.at[sl, j])

    def out_cp(step, sl, j):
        return pltpu.make_async_copy(
            slab.at[sl, pl.ds(j * CK, CK), :],
            o_any.at[step, pl.ds(j * CK, CK), :],
            outsem.at[sl, j])

    @pl.when(b >= 2)
    def _():
        for j in range(K):
            out_cp(b - 2, slot, j).wait()

    for j in range(K):
        in_cp(b, slot, j).start()
    for j in range(K):
        in_cp(b, slot, j).wait()
    for j in range(K):
        out_cp(b, slot, j).start()

    @pl.when(b == nb - 1)
    def _():
        for j in range(K):
            out_cp(b, slot, j).wait()

    @pl.when((b == nb - 1) & (nb >= 2))
    def _():
        for j in range(K):
            out_cp(b - 1, 1 - slot, j).wait()


def kernel(x, w1_t, w2_t):
    B, C, H, W = x.shape
    HW = H * W
    xr = x.reshape(B, C, HW)
    out = pl.pallas_call(
        _mcopy,
        out_shape=jax.ShapeDtypeStruct((B, C, HW), x.dtype),
        grid=(B,),
        in_specs=[pl.BlockSpec(memory_space=pl.ANY)],
        out_specs=pl.BlockSpec(memory_space=pl.ANY),
        scratch_shapes=[
            pltpu.VMEM((2, C, HW), x.dtype),
            pltpu.SemaphoreType.DMA((2, K)),
            pltpu.SemaphoreType.DMA((2, K)),
        ],
        compiler_params=pltpu.CompilerParams(
            dimension_semantics=("arbitrary",),
        ),
    )(xr)
    return out.reshape(B, C, H, W)


# speedup vs baseline: 1.2316x; 1.2316x over previous
import functools

import jax
import jax.numpy as jnp
from jax import lax
from jax.experimental import pallas as pl
from jax.experimental.pallas import tpu as pltpu


def _se_kernel(x_ref, w1_ref, w2_ref, o_ref, *, inv_hw):
    xf = x_ref[...]
    pooled = jnp.sum(xf, axis=-1, keepdims=True) * inv_hw
    for i in range(xf.shape[0]):
        y = pooled[i]
        h = lax.dot_general(w1_ref[...], y, (((1,), (0,)), ((), ())),
                            preferred_element_type=jnp.float32)
        h = jnp.maximum(h, 0.0)
        logits = lax.dot_general(w2_ref[...], h, (((1,), (0,)), ((), ())),
                                 preferred_element_type=jnp.float32)
        gates = jax.nn.sigmoid(logits)
        o_ref[i] = (xf[i] * gates).astype(o_ref.dtype)


def kernel(x, w1_t, w2_t):
    B, C, H, W = x.shape
    Cr = w1_t.shape[1]
    HW = H * W
    xr = x.reshape(B, C, HW)
    w1 = w1_t.T.astype(jnp.float32)
    w2 = w2_t.T.astype(jnp.float32)

    slab_bytes = C * ((HW + 127) // 128 * 128) * x.dtype.itemsize
    BB = 1
    for cand in (8, 4, 2):
        if B % cand == 0 and 4 * cand * slab_bytes <= 48 << 20:
            BB = cand
            break

    out = pl.pallas_call(
        functools.partial(_se_kernel, inv_hw=1.0 / float(HW)),
        out_shape=jax.ShapeDtypeStruct((B, C, HW), x.dtype),
        grid=(B // BB,),
        in_specs=[
            pl.BlockSpec((BB, C, HW), lambda b: (b, 0, 0)),
            pl.BlockSpec((Cr, C), lambda b: (0, 0)),
            pl.BlockSpec((C, Cr), lambda b: (0, 0)),
        ],
        out_specs=pl.BlockSpec((BB, C, HW), lambda b: (b, 0, 0)),
        compiler_params=pltpu.CompilerParams(
            dimension_semantics=("parallel",),
        ),
        cost_estimate=pl.CostEstimate(
            flops=2 * B * C * HW + 4 * B * C * Cr,
            transcendentals=B * C,
            bytes_accessed=2 * B * C * HW * x.dtype.itemsize,
        ),
    )(xr, w1, w2)
    return out.reshape(B, C, H, W)
